# baseline (speedup 1.0000x reference)
import functools
import jax
import jax.numpy as jnp
from jax import lax
from jax.experimental import pallas as pl
from jax.experimental.pallas import tpu as pltpu
from jax.experimental.pallas import tpu_sc as plsc

N = 10000
E = 160000
D = 128
R = 8

NC = 2
NS = 16
NW = NC * NS
CH = 128
CPT = 40
EPT = CH * CPT
EP = EPT * NW
NA = 10240
RPT = NA // NS

N2 = 10240
BN = 512


def _t1_body(x_ref, w_ref, sw_ref, aw_ref, y_ref, s13_ref):
    xb = x_ref[...]
    a = aw_ref[...].reshape(3, D)
    v = jnp.dot(a, sw_ref[...], preferred_element_type=jnp.float32)
    sT = lax.dot_general(v, xb, (((1,), (1,)), ((), ())),
                         preferred_element_type=jnp.float32)
    u1 = lax.bitcast_convert_type(sT[0:1].astype(jnp.bfloat16),
                                  jnp.uint16).astype(jnp.uint32)
    u3 = lax.bitcast_convert_type(sT[2:3].astype(jnp.bfloat16),
                                  jnp.uint16).astype(jnp.uint32)
    s13_ref[...] = lax.bitcast_convert_type(u1 | (u3 << 16), jnp.int32)
    for r in range(R):
        y_ref[:, r, :] = jnp.dot(xb, w_ref[r], preferred_element_type=jnp.float32)


def _t1(x, weight, shared_W, attn_W):
    return pl.pallas_call(
        _t1_body,
        grid=(N2 // BN,),
        in_specs=[
            pl.BlockSpec((BN, D), lambda i: (i, 0)),
            pl.BlockSpec((R, D, D), lambda i: (0, 0, 0)),
            pl.BlockSpec((D, D), lambda i: (0, 0)),
            pl.BlockSpec((1, 3 * D), lambda i: (0, 0)),
        ],
        out_specs=[
            pl.BlockSpec((BN, R, D), lambda i: (i, 0, 0)),
            pl.BlockSpec((1, BN), lambda i: (0, i)),
        ],
        out_shape=[
            jax.ShapeDtypeStruct((N2, R, D), jnp.float32),
            jax.ShapeDtypeStruct((1, N2), jnp.int32),
        ],
    )(x, weight, shared_W, attn_W)


_EROWS = E // CH
_MROWS = EP // CH
_T2_B = 8


def _t2m_body(ea_ref, ei_ref, rel_ref, sw_ref, aw_ref, m_ref):
    a = aw_ref[...].reshape(3, D)
    v = jnp.dot(a, sw_ref[...], preferred_element_type=jnp.float32)
    v2 = v[1]
    t = jnp.sum(ea_ref[...] * v2[None, None, :], axis=-1)
    src = ei_ref[0]
    dstv = ei_ref[1]
    gid = src * R + rel_ref[...]
    row0 = pl.program_id(0) * _T2_B
    rows = row0 + lax.broadcasted_iota(jnp.int32, (_T2_B, CH), 0)
    padm = rows >= _EROWS
    gid = jnp.where(padm, 0, gid)
    dstv = jnp.where(padm, N, dstv)
    m_ref[...] = jnp.stack(
        [gid, dstv, lax.bitcast_convert_type(t, jnp.int32)], axis=1)


def _t2m(edge_attr, edge_index, rel_type, shared_W, attn_W):
    ea3 = edge_attr.reshape(_EROWS, CH, D)
    ei3 = edge_index.reshape(2, _EROWS, CH)
    rel2 = rel_type.reshape(_EROWS, CH)
    return pl.pallas_call(
        _t2m_body,
        grid=(_MROWS // _T2_B,),
        in_specs=[
            pl.BlockSpec((_T2_B, CH, D),
                         lambda i: (jnp.minimum(i, _EROWS // _T2_B), 0, 0)),
            pl.BlockSpec((2, _T2_B, CH),
                         lambda i: (0, jnp.minimum(i, _EROWS // _T2_B), 0)),
            pl.BlockSpec((_T2_B, CH),
                         lambda i: (jnp.minimum(i, _EROWS // _T2_B), 0)),
            pl.BlockSpec((D, D), lambda i: (0, 0)),
            pl.BlockSpec((1, 3 * D), lambda i: (0, 0)),
        ],
        out_specs=pl.BlockSpec((_T2_B, 3, CH), lambda i: (i, 0, 0)),
        out_shape=jax.ShapeDtypeStruct((_MROWS, 3, CH), jnp.int32),
    )(ea3, ei3, rel2, shared_W, attn_W)


def _lane_ids():
    return lax.iota(jnp.int32, 16)


def _sc_body(table_hbm, meta_hbm, s13_hbm, out_hbm,
             mbufA, mbufB, s13_v, rowsA, rowsB, accum, semG, semM):
    cid = lax.axis_index("c")
    sid = lax.axis_index("s")
    wid = sid * NC + cid
    mbase = wid * CPT

    pltpu.sync_copy(s13_hbm, s13_v)

    zero = jnp.zeros((16,), jnp.float32)

    def _zrow(i, _):
        for j in range(D // 16):
            rowsA[i, pl.ds(j * 16, 16)] = zero
        return 0

    lax.fori_loop(0, CH, _zrow, 0)
    for k in range(RPT // CH):
        pltpu.sync_copy(rowsA, accum.at[pl.ds(sid * RPT + k * CH, CH)])
    plsc.subcore_barrier()

    pltpu.sync_copy(meta_hbm.at[mbase], mbufA)
    pltpu.async_copy(table_hbm.at[mbufA.at[0]], rowsA, semG)
    pltpu.async_copy(meta_hbm.at[mbase + 1], mbufB, semM)

    def _do_chunk(mbuf, rows):
        def _grp(g, _):
            gv = mbuf[0, pl.ds(g * 16, 16)]
            dstv = mbuf[1, pl.ds(g * 16, 16)]
            tv = plsc.bitcast(mbuf[2, pl.ds(g * 16, 16)], jnp.float32)
            p1 = plsc.load_gather(s13_v, [lax.shift_right_logical(gv, 3)])
            p3 = plsc.load_gather(s13_v, [dstv])
            ev = (plsc.bitcast(lax.shift_left(p1, 16), jnp.float32) + tv +
                  plsc.bitcast(jnp.bitwise_and(p3, jnp.int32(-65536)),
                               jnp.float32))
            for l in range(16):
                es = ev[jnp.full((16,), l, jnp.int32)]
                i = g * 16 + l
                for j in range(D // 16):
                    rows[i, pl.ds(j * 16, 16)] = rows[i, pl.ds(j * 16, 16)] * es
            return 0

        lax.fori_loop(0, CH // 16, _grp, 0)
        pltpu.sync_copy(rows, accum.at[mbuf.at[1]], add=True)

    def _iter(c, mbuf, rows, mbuf_n, rows_n):
        pltpu.make_async_copy(table_hbm.at[mbuf.at[0]], rows, semG).wait()

        @pl.when(c + 1 < CPT)
        def _():
            pltpu.make_async_copy(meta_hbm.at[mbase], mbuf_n, semM).wait()
            pltpu.async_copy(table_hbm.at[mbuf_n.at[0]], rows_n, semG)

        _do_chunk(mbuf, rows)

        @pl.when(c + 2 < CPT)
        def _():
            pltpu.async_copy(meta_hbm.at[mbase + c + 2], mbuf, semM)

    def _pair(cc, _):
        _iter(2 * cc, mbufA, rowsA, mbufB, rowsB)
        _iter(2 * cc + 1, mbufB, rowsB, mbufA, rowsA)
        return 0

    lax.fori_loop(0, CPT // 2, _pair, 0)
    plsc.subcore_barrier()

    pltpu.sync_copy(accum.at[pl.ds(sid * RPT, RPT)],
                    out_hbm.at[cid, pl.ds(sid * RPT, RPT)])


def _sc(table, meta, s13):
    mesh = plsc.VectorSubcoreMesh(core_axis_name="c", subcore_axis_name="s")
    f = functools.partial(
        pl.kernel,
        out_type=jax.ShapeDtypeStruct((NC, NA, D), jnp.float32),
        mesh=mesh,
        scratch_types=[
            pltpu.VMEM((3, CH), jnp.int32),
            pltpu.VMEM((3, CH), jnp.int32),
            pltpu.VMEM((N2,), jnp.int32),
            pltpu.VMEM((CH, D), jnp.float32),
            pltpu.VMEM((CH, D), jnp.float32),
            pltpu.VMEM_SHARED((NA, D), jnp.float32),
            pltpu.SemaphoreType.DMA,
            pltpu.SemaphoreType.DMA,
        ],
        compiler_params=pltpu.CompilerParams(needs_layout_passes=False),
    )(_sc_body)
    return f(table, meta, s13)


def _t3_body(p0_ref, p1_ref, o_ref):
    o_ref[...] = p0_ref[...] + p1_ref[...]


def _t3(p0, p1):
    bn = 400
    return pl.pallas_call(
        _t3_body,
        grid=(N // bn,),
        in_specs=[pl.BlockSpec((bn, D), lambda i: (i, 0)),
                  pl.BlockSpec((bn, D), lambda i: (i, 0))],
        out_specs=pl.BlockSpec((bn, D), lambda i: (i, 0)),
        out_shape=jax.ShapeDtypeStruct((N, D), jnp.float32),
    )(p0, p1)


def kernel(x, edge_index, edge_attr, rel_type, weight, shared_W, attn_W):
    xp = jnp.concatenate([x, jnp.zeros((N2 - N, D), jnp.float32)])
    y, s13_2d = _t1(xp, weight, shared_W, attn_W)
    table = y.reshape(N2 * R, D)
    s13 = s13_2d.reshape(N2)

    meta = _t2m(edge_attr, edge_index, rel_type, shared_W, attn_W)

    partial = _sc(table, meta, s13)
    return _t3(partial[0], partial[1])


# speedup vs baseline: 1.0598x; 1.0598x over previous
import functools
import jax
import jax.numpy as jnp
from jax import lax
from jax.experimental import pallas as pl
from jax.experimental.pallas import tpu as pltpu
from jax.experimental.pallas import tpu_sc as plsc

N = 10000
E = 160000
D = 128
R = 8

NC = 2
NS = 16
NW = NC * NS
CH = 128
CPT = 40
EPT = CH * CPT
EP = EPT * NW
NA = 10240
RPT = NA // NS

N2 = 10240
BN = 512


def _t1_body(x_ref, wc_ref, sw_ref, aw_ref, y_ref, s13_ref):
    xb = x_ref[...]
    a = aw_ref[...].reshape(3, D)
    v = jnp.dot(a, sw_ref[...], preferred_element_type=jnp.float32)
    sT = lax.dot_general(v, xb, (((1,), (1,)), ((), ())),
                         preferred_element_type=jnp.float32)
    u1 = lax.bitcast_convert_type(sT[0:1].astype(jnp.bfloat16),
                                  jnp.uint16).astype(jnp.uint32)
    u3 = lax.bitcast_convert_type(sT[2:3].astype(jnp.bfloat16),
                                  jnp.uint16).astype(jnp.uint32)
    s13_ref[...] = lax.bitcast_convert_type(u1 | (u3 << 16), jnp.int32)
    y_ref[...] = jnp.dot(xb, wc_ref[...], preferred_element_type=jnp.float32)


def _t1(x, weight, shared_W, attn_W):
    return pl.pallas_call(
        _t1_body,
        grid=(N2 // BN,),
        in_specs=[
            pl.BlockSpec((BN, D), lambda i: (i, 0)),
            pl.BlockSpec((D, R * D), lambda i: (0, 0)),
            pl.BlockSpec((D, D), lambda i: (0, 0)),
            pl.BlockSpec((1, 3 * D), lambda i: (0, 0)),
        ],
        out_specs=[
            pl.BlockSpec((BN, R * D), lambda i: (i, 0)),
            pl.BlockSpec((1, BN), lambda i: (0, i)),
        ],
        out_shape=[
            jax.ShapeDtypeStruct((N2, R * D), jnp.float32),
            jax.ShapeDtypeStruct((1, N2), jnp.int32),
        ],
    )(x, jnp.concatenate([weight[r] for r in range(R)], axis=1),
      shared_W, attn_W)


_T2_ROWS = 625
_T2_OUT_ROWS = EP // 256
_T2_B = 8


def _t2_body(ea_ref, sw_ref, aw_ref, t_ref):
    a = aw_ref[...].reshape(3, D)
    v = jnp.dot(a, sw_ref[...], preferred_element_type=jnp.float32)
    v2 = v[1]
    eb = ea_ref[...]
    t_ref[...] = jnp.sum(eb * v2[None, None, :], axis=-1)


def _t2(edge_attr, shared_W, attn_W):
    ea3 = edge_attr.reshape(_T2_ROWS, 256, D)
    return pl.pallas_call(
        _t2_body,
        grid=(pl.cdiv(_T2_ROWS, _T2_B),),
        in_specs=[
            pl.BlockSpec((_T2_B, 256, D), lambda i: (i, 0, 0)),
            pl.BlockSpec((D, D), lambda i: (0, 0)),
            pl.BlockSpec((1, 3 * D), lambda i: (0, 0)),
        ],
        out_specs=pl.BlockSpec((_T2_B, 256), lambda i: (i, 0)),
        out_shape=jax.ShapeDtypeStruct((_T2_OUT_ROWS, 256), jnp.float32),
    )(ea3, shared_W, attn_W)


def _lane_ids():
    return lax.iota(jnp.int32, 16)


def _sc_body(table_hbm, meta_hbm, s13_hbm, out_hbm,
             mbufA, mbufB, s13_v, rowsA, rowsB, accum, semG, semM):
    cid = lax.axis_index("c")
    sid = lax.axis_index("s")
    wid = sid * NC + cid
    mbase = wid * CPT

    pltpu.sync_copy(s13_hbm, s13_v)

    zero = jnp.zeros((16,), jnp.float32)

    def _zrow(i, _):
        for j in range(D // 16):
            rowsA[i, pl.ds(j * 16, 16)] = zero
        return 0

    lax.fori_loop(0, CH, _zrow, 0)
    for k in range(RPT // CH):
        pltpu.sync_copy(rowsA, accum.at[pl.ds(sid * RPT + k * CH, CH)])
    plsc.subcore_barrier()

    pltpu.sync_copy(meta_hbm.at[mbase], mbufA)
    pltpu.async_copy(table_hbm.at[mbufA.at[0]], rowsA, semG)
    pltpu.async_copy(meta_hbm.at[mbase + 1], mbufB, semM)

    def _do_chunk(mbuf, rows):
        def _grp(g, _):
            gv = mbuf[0, pl.ds(g * 16, 16)]
            dstv = mbuf[1, pl.ds(g * 16, 16)]
            tv = plsc.bitcast(mbuf[2, pl.ds(g * 16, 16)], jnp.float32)
            p1 = plsc.load_gather(s13_v, [lax.shift_right_logical(gv, 3)])
            p3 = plsc.load_gather(s13_v, [dstv])
            ev = (plsc.bitcast(lax.shift_left(p1, 16), jnp.float32) + tv +
                  plsc.bitcast(jnp.bitwise_and(p3, jnp.int32(-65536)),
                               jnp.float32))
            for l in range(16):
                es = ev[jnp.full((16,), l, jnp.int32)]
                i = g * 16 + l
                for j in range(D // 16):
                    rows[i, pl.ds(j * 16, 16)] = rows[i, pl.ds(j * 16, 16)] * es
            return 0

        lax.fori_loop(0, CH // 16, _grp, 0)
        pltpu.sync_copy(rows, accum.at[mbuf.at[1]], add=True)

    def _iter(c, mbuf, rows, mbuf_n, rows_n):
        pltpu.make_async_copy(table_hbm.at[mbuf.at[0]], rows, semG).wait()

        @pl.when(c + 1 < CPT)
        def _():
            pltpu.make_async_copy(meta_hbm.at[mbase], mbuf_n, semM).wait()
            pltpu.async_copy(table_hbm.at[mbuf_n.at[0]], rows_n, semG)

        _do_chunk(mbuf, rows)

        @pl.when(c + 2 < CPT)
        def _():
            pltpu.async_copy(meta_hbm.at[mbase + c + 2], mbuf, semM)

    def _pair(cc, _):
        _iter(2 * cc, mbufA, rowsA, mbufB, rowsB)
        _iter(2 * cc + 1, mbufB, rowsB, mbufA, rowsA)
        return 0

    lax.fori_loop(0, CPT // 2, _pair, 0)
    plsc.subcore_barrier()

    pltpu.sync_copy(accum.at[pl.ds(sid * RPT, RPT)],
                    out_hbm.at[cid, pl.ds(sid * RPT, RPT)])


def _sc(table, meta, s13):
    mesh = plsc.VectorSubcoreMesh(core_axis_name="c", subcore_axis_name="s")
    f = functools.partial(
        pl.kernel,
        out_type=jax.ShapeDtypeStruct((NC, NA, D), jnp.float32),
        mesh=mesh,
        scratch_types=[
            pltpu.VMEM((3, CH), jnp.int32),
            pltpu.VMEM((3, CH), jnp.int32),
            pltpu.VMEM((N2,), jnp.int32),
            pltpu.VMEM((CH, D), jnp.float32),
            pltpu.VMEM((CH, D), jnp.float32),
            pltpu.VMEM_SHARED((NA, D), jnp.float32),
            pltpu.SemaphoreType.DMA,
            pltpu.SemaphoreType.DMA,
        ],
        compiler_params=pltpu.CompilerParams(needs_layout_passes=False),
    )(_sc_body)
    return f(table, meta, s13)


def _t3_body(p0_ref, p1_ref, o_ref):
    o_ref[...] = p0_ref[...] + p1_ref[...]


def _t3(p0, p1):
    bn = 400
    return pl.pallas_call(
        _t3_body,
        grid=(N // bn,),
        in_specs=[pl.BlockSpec((bn, D), lambda i: (i, 0)),
                  pl.BlockSpec((bn, D), lambda i: (i, 0))],
        out_specs=pl.BlockSpec((bn, D), lambda i: (i, 0)),
        out_shape=jax.ShapeDtypeStruct((N, D), jnp.float32),
    )(p0, p1)


def kernel(x, edge_index, edge_attr, rel_type, weight, shared_W, attn_W):
    src = edge_index[0]
    dst = edge_index[1]

    xp = jnp.concatenate([x, jnp.zeros((N2 - N, D), jnp.float32)])
    y, s13_2d = _t1(xp, weight, shared_W, attn_W)
    table = y.reshape(N2 * R, D)
    s13 = s13_2d.reshape(N2)

    t = _t2(edge_attr, shared_W, attn_W).reshape(EP)

    pad = EP - E
    gidx = src * R + rel_type
    gidx2 = jnp.concatenate([gidx, jnp.zeros((pad,), jnp.int32)]).reshape(
        NW * CPT, CH)
    dst2 = jnp.concatenate([dst, jnp.full((pad,), N, jnp.int32)]).reshape(
        NW * CPT, CH)
    tbits = lax.bitcast_convert_type(t, jnp.int32).reshape(NW * CPT, CH)
    meta = jnp.stack([gidx2, dst2, tbits], axis=1)

    partial = _sc(table, meta, s13)
    return _t3(partial[0], partial[1])


# speedup vs baseline: 1.0939x; 1.0321x over previous
import functools
import jax
import jax.numpy as jnp
from jax import lax
from jax.experimental import pallas as pl
from jax.experimental.pallas import tpu as pltpu
from jax.experimental.pallas import tpu_sc as plsc

N = 10000
E = 160000
D = 128
R = 8

NC = 2
NS = 16
NW = NC * NS
CH = 128
CPT = 40
EPT = CH * CPT
EP = EPT * NW
NA = 10240
RPT = NA // NS

BN = 400


def _t1_body(x_ref, w_ref, sw_ref, aw_ref, y_ref, s_ref):
    xb = x_ref[...]
    a = aw_ref[...].reshape(3, D)
    v = jnp.dot(a, sw_ref[...], preferred_element_type=jnp.float32)
    vpad = jnp.concatenate([v, jnp.zeros((D - 3, D), jnp.float32)], axis=0)
    s_ref[...] = jnp.dot(xb, vpad.T, preferred_element_type=jnp.float32)
    for r in range(R):
        y_ref[:, r, :] = jnp.dot(xb, w_ref[r], preferred_element_type=jnp.float32)


def _t1(x, weight, shared_W, attn_W):
    return pl.pallas_call(
        _t1_body,
        grid=(N // BN,),
        in_specs=[
            pl.BlockSpec((BN, D), lambda i: (i, 0)),
            pl.BlockSpec((R, D, D), lambda i: (0, 0, 0)),
            pl.BlockSpec((D, D), lambda i: (0, 0)),
            pl.BlockSpec((1, 3 * D), lambda i: (0, 0)),
        ],
        out_specs=[
            pl.BlockSpec((BN, R, D), lambda i: (i, 0, 0)),
            pl.BlockSpec((BN, D), lambda i: (i, 0)),
        ],
        out_shape=[
            jax.ShapeDtypeStruct((N, R, D), jnp.float32),
            jax.ShapeDtypeStruct((N, D), jnp.float32),
        ],
    )(x, weight, shared_W, attn_W)


_T2_ROWS = 625
_T2_OUT_ROWS = EP // 256
_T2_B = 8


def _t2_body(ea_ref, sw_ref, aw_ref, t_ref):
    a = aw_ref[...].reshape(3, D)
    v = jnp.dot(a, sw_ref[...], preferred_element_type=jnp.float32)
    v2 = v[1]
    eb = ea_ref[...]
    t_ref[...] = jnp.sum(eb * v2[None, None, :], axis=-1)


def _t2(edge_attr, shared_W, attn_W):
    ea3 = edge_attr.reshape(_T2_ROWS, 256, D)
    return pl.pallas_call(
        _t2_body,
        grid=(pl.cdiv(_T2_ROWS, _T2_B),),
        in_specs=[
            pl.BlockSpec((_T2_B, 256, D), lambda i: (i, 0, 0)),
            pl.BlockSpec((D, D), lambda i: (0, 0)),
            pl.BlockSpec((1, 3 * D), lambda i: (0, 0)),
        ],
        out_specs=pl.BlockSpec((_T2_B, 256), lambda i: (i, 0)),
        out_shape=jax.ShapeDtypeStruct((_T2_OUT_ROWS, 256), jnp.float32),
    )(ea3, shared_W, attn_W)


def _sc_body(table_hbm, meta_hbm, s13_hbm, out_hbm,
             mbufA, mbufB, s13_v, rowsA, rowsB, accum, semG, semM):
    cid = lax.axis_index("c")
    sid = lax.axis_index("s")
    wid = sid * NC + cid
    mbase = wid * CPT

    pltpu.sync_copy(s13_hbm, s13_v)

    zero = jnp.zeros((16,), jnp.float32)

    def _zrow(i, _):
        for j in range(D // 16):
            rowsA[i, pl.ds(j * 16, 16)] = zero
        return 0

    lax.fori_loop(0, CH, _zrow, 0)
    for k in range(RPT // CH):
        pltpu.sync_copy(rowsA, accum.at[pl.ds(sid * RPT + k * CH, CH)])
    plsc.subcore_barrier()

    pltpu.sync_copy(meta_hbm.at[mbase], mbufA)
    pltpu.async_copy(table_hbm.at[mbufA.at[0]], rowsA, semG)
    pltpu.async_copy(meta_hbm.at[mbase + 1], mbufB, semM)

    def _do_chunk(mbuf, rows):
        def _grp(g, _):
            gv = mbuf[0, pl.ds(g * 16, 16)]
            dstv = mbuf[1, pl.ds(g * 16, 16)]
            tv = plsc.bitcast(mbuf[2, pl.ds(g * 16, 16)], jnp.float32)
            p1 = plsc.load_gather(s13_v, [lax.shift_right_logical(gv, 3)])
            p3 = plsc.load_gather(s13_v, [dstv])
            ev = (plsc.bitcast(lax.shift_left(p1, 16), jnp.float32) + tv +
                  plsc.bitcast(jnp.bitwise_and(p3, jnp.int32(-65536)),
                               jnp.float32))
            for l in range(16):
                es = ev[jnp.full((16,), l, jnp.int32)]
                i = g * 16 + l
                for j in range(D // 16):
                    rows[i, pl.ds(j * 16, 16)] = rows[i, pl.ds(j * 16, 16)] * es
            return 0

        lax.fori_loop(0, CH // 16, _grp, 0)
        pltpu.sync_copy(rows, accum.at[mbuf.at[1]], add=True)

    def _iter(c, mbuf, rows, mbuf_n, rows_n):
        pltpu.make_async_copy(table_hbm.at[mbuf.at[0]], rows, semG).wait()

        @pl.when(c + 1 < CPT)
        def _():
            pltpu.make_async_copy(meta_hbm.at[mbase], mbuf_n, semM).wait()
            pltpu.async_copy(table_hbm.at[mbuf_n.at[0]], rows_n, semG)

        _do_chunk(mbuf, rows)

        @pl.when(c + 2 < CPT)
        def _():
            pltpu.async_copy(meta_hbm.at[mbase + c + 2], mbuf, semM)

    def _pair(cc, _):
        _iter(2 * cc, mbufA, rowsA, mbufB, rowsB)
        _iter(2 * cc + 1, mbufB, rowsB, mbufA, rowsA)
        return 0

    lax.fori_loop(0, CPT // 2, _pair, 0)
    plsc.subcore_barrier()

    pltpu.sync_copy(accum.at[pl.ds(sid * RPT, RPT)],
                    out_hbm.at[cid, pl.ds(sid * RPT, RPT)])


def _sc(table, meta, s13):
    mesh = plsc.VectorSubcoreMesh(core_axis_name="c", subcore_axis_name="s")
    f = functools.partial(
        pl.kernel,
        out_type=jax.ShapeDtypeStruct((NC, NA, D), jnp.float32),
        mesh=mesh,
        scratch_types=[
            pltpu.VMEM((3, CH), jnp.int32),
            pltpu.VMEM((3, CH), jnp.int32),
            pltpu.VMEM((N,), jnp.int32),
            pltpu.VMEM((CH, D), jnp.float32),
            pltpu.VMEM((CH, D), jnp.float32),
            pltpu.VMEM_SHARED((NA, D), jnp.float32),
            pltpu.SemaphoreType.DMA,
            pltpu.SemaphoreType.DMA,
        ],
        compiler_params=pltpu.CompilerParams(needs_layout_passes=False),
    )(_sc_body)
    return f(table, meta, s13)


def _t3_body(p0_ref, p1_ref, o_ref):
    o_ref[...] = p0_ref[...] + p1_ref[...]


def _t3(p0, p1):
    return pl.pallas_call(
        _t3_body,
        grid=(N // BN,),
        in_specs=[pl.BlockSpec((BN, D), lambda i: (i, 0)),
                  pl.BlockSpec((BN, D), lambda i: (i, 0))],
        out_specs=pl.BlockSpec((BN, D), lambda i: (i, 0)),
        out_shape=jax.ShapeDtypeStruct((N, D), jnp.float32),
    )(p0, p1)


def kernel(x, edge_index, edge_attr, rel_type, weight, shared_W, attn_W):
    src = edge_index[0]
    dst = edge_index[1]

    y, s = _t1(x, weight, shared_W, attn_W)
    table = y.reshape(N * R, D)
    s1u = lax.bitcast_convert_type(s[:, 0].astype(jnp.bfloat16),
                                   jnp.uint16).astype(jnp.uint32)
    s3u = lax.bitcast_convert_type(s[:, 2].astype(jnp.bfloat16),
                                   jnp.uint16).astype(jnp.uint32)
    s13 = lax.bitcast_convert_type(s1u | (s3u << 16), jnp.int32)

    t = _t2(edge_attr, shared_W, attn_W).reshape(EP)

    pad = EP - E
    gidx = src * R + rel_type
    gidx2 = jnp.concatenate([gidx, jnp.zeros((pad,), jnp.int32)]).reshape(
        NW * CPT, CH)
    dst2 = jnp.concatenate([dst, jnp.full((pad,), N, jnp.int32)]).reshape(
        NW * CPT, CH)
    tbits = lax.bitcast_convert_type(t, jnp.int32).reshape(NW * CPT, CH)
    meta = jnp.stack([gidx2, dst2, tbits], axis=1)

    partial = _sc(table, meta, s13)
    return _t3(partial[0], partial[1])


# speedup vs baseline: 1.0942x; 1.0003x over previous
import functools
import jax
import jax.numpy as jnp
from jax import lax
from jax.experimental import pallas as pl
from jax.experimental.pallas import tpu as pltpu
from jax.experimental.pallas import tpu_sc as plsc

N = 10000
E = 160000
D = 128
R = 8

NC = 2
NS = 16
NW = NC * NS
CH = 128
CPT = 40
EPT = CH * CPT
EP = EPT * NW
NA = 10240
RPT = NA // NS

BN = 400


def _t1_body(x_ref, w_ref, sw_ref, aw_ref, y_ref, s_ref):
    xb = x_ref[...]
    a = aw_ref[...].reshape(3, D)
    v = jnp.dot(a, sw_ref[...], preferred_element_type=jnp.float32)
    vpad = jnp.concatenate([v, jnp.zeros((D - 3, D), jnp.float32)], axis=0)
    s_ref[...] = jnp.dot(xb, vpad.T, preferred_element_type=jnp.float32)
    for r in range(R):
        y_ref[:, r, :] = jnp.dot(xb, w_ref[r], preferred_element_type=jnp.float32)


def _t1(x, weight, shared_W, attn_W):
    return pl.pallas_call(
        _t1_body,
        grid=(N // BN,),
        in_specs=[
            pl.BlockSpec((BN, D), lambda i: (i, 0)),
            pl.BlockSpec((R, D, D), lambda i: (0, 0, 0)),
            pl.BlockSpec((D, D), lambda i: (0, 0)),
            pl.BlockSpec((1, 3 * D), lambda i: (0, 0)),
        ],
        out_specs=[
            pl.BlockSpec((BN, R, D), lambda i: (i, 0, 0)),
            pl.BlockSpec((BN, D), lambda i: (i, 0)),
        ],
        out_shape=[
            jax.ShapeDtypeStruct((N, R, D), jnp.float32),
            jax.ShapeDtypeStruct((N, D), jnp.float32),
        ],
    )(x, weight, shared_W, attn_W)


_T2_ROWS = 625
_T2_OUT_ROWS = EP // 256
_T2_B = 8


def _t2_body(ea_ref, sw_ref, aw_ref, t_ref):
    a = aw_ref[...].reshape(3, D)
    v = jnp.dot(a, sw_ref[...], preferred_element_type=jnp.float32)
    v2 = v[1]
    eb = ea_ref[...]
    t_ref[...] = jnp.sum(eb * v2[None, None, :], axis=-1)


def _t2(edge_attr, shared_W, attn_W):
    ea3 = edge_attr.reshape(_T2_ROWS, 256, D)
    return pl.pallas_call(
        _t2_body,
        grid=(pl.cdiv(_T2_ROWS, _T2_B),),
        in_specs=[
            pl.BlockSpec((_T2_B, 256, D), lambda i: (i, 0, 0)),
            pl.BlockSpec((D, D), lambda i: (0, 0)),
            pl.BlockSpec((1, 3 * D), lambda i: (0, 0)),
        ],
        out_specs=pl.BlockSpec((_T2_B, 256), lambda i: (i, 0)),
        out_shape=jax.ShapeDtypeStruct((_T2_OUT_ROWS, 256), jnp.float32),
    )(ea3, shared_W, attn_W)


def _sc_body(table_hbm, meta_hbm, s13_hbm, out_hbm,
             mbufA, mbufB, s13_v, rowsA, rowsB, accum, semG, semM):
    cid = lax.axis_index("c")
    sid = lax.axis_index("s")
    wid = sid * NC + cid
    mbase = wid * CPT

    pltpu.sync_copy(s13_hbm, s13_v)

    zero = jnp.zeros((16,), jnp.float32)

    def _zrow(i, _):
        for j in range(D // 16):
            rowsA[i, pl.ds(j * 16, 16)] = zero
        return 0

    lax.fori_loop(0, CH, _zrow, 0)
    for k in range(RPT // CH):
        pltpu.sync_copy(rowsA, accum.at[pl.ds(sid * RPT + k * CH, CH)])
    plsc.subcore_barrier()

    pltpu.sync_copy(meta_hbm.at[mbase], mbufA)
    pltpu.async_copy(table_hbm.at[mbufA.at[0]], rowsA, semG)
    pltpu.async_copy(meta_hbm.at[mbase + 1], mbufB, semM)

    def _do_chunk(mbuf, rows):
        @plsc.parallel_loop(0, CH // 16, unroll=2)
        def _grp(g):
            gv = mbuf[0, pl.ds(g * 16, 16)]
            dstv = mbuf[1, pl.ds(g * 16, 16)]
            tv = plsc.bitcast(mbuf[2, pl.ds(g * 16, 16)], jnp.float32)
            p1 = plsc.load_gather(s13_v, [lax.shift_right_logical(gv, 3)])
            p3 = plsc.load_gather(s13_v, [dstv])
            ev = (plsc.bitcast(lax.shift_left(p1, 16), jnp.float32) + tv +
                  plsc.bitcast(jnp.bitwise_and(p3, jnp.int32(-65536)),
                               jnp.float32))
            for l in range(16):
                es = ev[jnp.full((16,), l, jnp.int32)]
                i = g * 16 + l
                for j in range(D // 16):
                    rows[i, pl.ds(j * 16, 16)] = rows[i, pl.ds(j * 16, 16)] * es

        pltpu.sync_copy(rows, accum.at[mbuf.at[1]], add=True)

    def _iter(c, mbuf, rows, mbuf_n, rows_n):
        pltpu.make_async_copy(table_hbm.at[mbuf.at[0]], rows, semG).wait()

        @pl.when(c + 1 < CPT)
        def _():
            pltpu.make_async_copy(meta_hbm.at[mbase], mbuf_n, semM).wait()
            pltpu.async_copy(table_hbm.at[mbuf_n.at[0]], rows_n, semG)

        _do_chunk(mbuf, rows)

        @pl.when(c + 2 < CPT)
        def _():
            pltpu.async_copy(meta_hbm.at[mbase + c + 2], mbuf, semM)

    def _pair(cc, _):
        _iter(2 * cc, mbufA, rowsA, mbufB, rowsB)
        _iter(2 * cc + 1, mbufB, rowsB, mbufA, rowsA)
        return 0

    lax.fori_loop(0, CPT // 2, _pair, 0)
    plsc.subcore_barrier()

    pltpu.sync_copy(accum.at[pl.ds(sid * RPT, RPT)],
                    out_hbm.at[cid, pl.ds(sid * RPT, RPT)])


def _sc(table, meta, s13):
    mesh = plsc.VectorSubcoreMesh(core_axis_name="c", subcore_axis_name="s")
    f = functools.partial(
        pl.kernel,
        out_type=jax.ShapeDtypeStruct((NC, NA, D), jnp.float32),
        mesh=mesh,
        scratch_types=[
            pltpu.VMEM((3, CH), jnp.int32),
            pltpu.VMEM((3, CH), jnp.int32),
            pltpu.VMEM((N,), jnp.int32),
            pltpu.VMEM((CH, D), jnp.float32),
            pltpu.VMEM((CH, D), jnp.float32),
            pltpu.VMEM_SHARED((NA, D), jnp.float32),
            pltpu.SemaphoreType.DMA,
            pltpu.SemaphoreType.DMA,
        ],
        compiler_params=pltpu.CompilerParams(needs_layout_passes=False),
    )(_sc_body)
    return f(table, meta, s13)


def _t3_body(p0_ref, p1_ref, o_ref):
    o_ref[...] = p0_ref[...] + p1_ref[...]


def _t3(p0, p1):
    return pl.pallas_call(
        _t3_body,
        grid=(N // BN,),
        in_specs=[pl.BlockSpec((BN, D), lambda i: (i, 0)),
                  pl.BlockSpec((BN, D), lambda i: (i, 0))],
        out_specs=pl.BlockSpec((BN, D), lambda i: (i, 0)),
        out_shape=jax.ShapeDtypeStruct((N, D), jnp.float32),
    )(p0, p1)


def kernel(x, edge_index, edge_attr, rel_type, weight, shared_W, attn_W):
    src = edge_index[0]
    dst = edge_index[1]

    y, s = _t1(x, weight, shared_W, attn_W)
    table = y.reshape(N * R, D)
    s1u = lax.bitcast_convert_type(s[:, 0].astype(jnp.bfloat16),
                                   jnp.uint16).astype(jnp.uint32)
    s3u = lax.bitcast_convert_type(s[:, 2].astype(jnp.bfloat16),
                                   jnp.uint16).astype(jnp.uint32)
    s13 = lax.bitcast_convert_type(s1u | (s3u << 16), jnp.int32)

    t = _t2(edge_attr, shared_W, attn_W).reshape(EP)

    pad = EP - E
    gidx = src * R + rel_type
    gidx2 = jnp.concatenate([gidx, jnp.zeros((pad,), jnp.int32)]).reshape(
        NW * CPT, CH)
    dst2 = jnp.concatenate([dst, jnp.full((pad,), N, jnp.int32)]).reshape(
        NW * CPT, CH)
    tbits = lax.bitcast_convert_type(t, jnp.int32).reshape(NW * CPT, CH)
    meta = jnp.stack([gidx2, dst2, tbits], axis=1)

    partial = _sc(table, meta, s13)
    return _t3(partial[0], partial[1])


# speedup vs baseline: 1.1750x; 1.0739x over previous
import functools
import jax
import jax.numpy as jnp
from jax import lax
from jax.experimental import pallas as pl
from jax.experimental.pallas import tpu as pltpu
from jax.experimental.pallas import tpu_sc as plsc

N = 10000
E = 160000
D = 128
R = 8

NC = 2
NS = 16
NW = NC * NS
CH = 128
CPT = 40
EPT = CH * CPT
EP = EPT * NW
NA = 10240
RPT = NA // NS

BN = 400


def _t1_body(x_ref, w_ref, sw_ref, aw_ref, y_ref, s_ref):
    xb = x_ref[...]
    a = aw_ref[...].reshape(3, D)
    v = jnp.dot(a, sw_ref[...], preferred_element_type=jnp.float32)
    vpad = jnp.concatenate([v, jnp.zeros((D - 3, D), jnp.float32)], axis=0)
    s_ref[...] = jnp.dot(xb, vpad.T, preferred_element_type=jnp.float32)
    for r in range(R):
        y_ref[:, r, :] = jnp.dot(xb, w_ref[r], preferred_element_type=jnp.float32)


def _t1(x, weight, shared_W, attn_W):
    return pl.pallas_call(
        _t1_body,
        grid=(N // BN,),
        in_specs=[
            pl.BlockSpec((BN, D), lambda i: (i, 0)),
            pl.BlockSpec((R, D, D), lambda i: (0, 0, 0)),
            pl.BlockSpec((D, D), lambda i: (0, 0)),
            pl.BlockSpec((1, 3 * D), lambda i: (0, 0)),
        ],
        out_specs=[
            pl.BlockSpec((BN, R, D), lambda i: (i, 0, 0)),
            pl.BlockSpec((BN, D), lambda i: (i, 0)),
        ],
        out_shape=[
            jax.ShapeDtypeStruct((N, R, D), jnp.float32),
            jax.ShapeDtypeStruct((N, D), jnp.float32),
        ],
    )(x, weight, shared_W, attn_W)


_T2_ROWS = 625
_T2_OUT_ROWS = EP // 256
_T2_B = 24


def _t2_body(ea_ref, sw_ref, aw_ref, t_ref):
    a = aw_ref[...].reshape(3, D)
    v = jnp.dot(a, sw_ref[...], preferred_element_type=jnp.float32)
    v2 = v[1]
    eb = ea_ref[...]
    t_ref[...] = jnp.sum(eb * v2[None, None, :], axis=-1)


def _t2(edge_attr, shared_W, attn_W):
    ea3 = edge_attr.reshape(_T2_ROWS, 256, D)
    return pl.pallas_call(
        _t2_body,
        grid=(pl.cdiv(_T2_ROWS, _T2_B),),
        in_specs=[
            pl.BlockSpec((_T2_B, 256, D), lambda i: (i, 0, 0)),
            pl.BlockSpec((D, D), lambda i: (0, 0)),
            pl.BlockSpec((1, 3 * D), lambda i: (0, 0)),
        ],
        out_specs=pl.BlockSpec((_T2_B, 256), lambda i: (i, 0)),
        out_shape=jax.ShapeDtypeStruct((_T2_OUT_ROWS, 256), jnp.float32),
    )(ea3, shared_W, attn_W)


def _sc_body(table_hbm, meta_hbm, s13_hbm, out_hbm,
             mbufA, mbufB, s13_v, rowsA, rowsB, accum, semG, semM):
    cid = lax.axis_index("c")
    sid = lax.axis_index("s")
    wid = sid * NC + cid
    mbase = wid * CPT

    pltpu.sync_copy(s13_hbm, s13_v)

    zero = jnp.zeros((16,), jnp.float32)

    def _zrow(i, _):
        for j in range(D // 16):
            rowsA[i, pl.ds(j * 16, 16)] = zero
        return 0

    lax.fori_loop(0, CH, _zrow, 0)
    for k in range(RPT // CH):
        pltpu.sync_copy(rowsA, accum.at[pl.ds(sid * RPT + k * CH, CH)])
    plsc.subcore_barrier()

    pltpu.sync_copy(meta_hbm.at[mbase], mbufA)
    pltpu.async_copy(table_hbm.at[mbufA.at[0]], rowsA, semG)
    pltpu.async_copy(meta_hbm.at[mbase + 1], mbufB, semM)

    def _do_chunk(mbuf, rows):
        @plsc.parallel_loop(0, CH // 16, unroll=2)
        def _grp(g):
            gv = mbuf[0, pl.ds(g * 16, 16)]
            dstv = mbuf[1, pl.ds(g * 16, 16)]
            tv = plsc.bitcast(mbuf[2, pl.ds(g * 16, 16)], jnp.float32)
            p1 = plsc.load_gather(s13_v, [lax.shift_right_logical(gv, 3)])
            p3 = plsc.load_gather(s13_v, [dstv])
            ev = (plsc.bitcast(lax.shift_left(p1, 16), jnp.float32) + tv +
                  plsc.bitcast(jnp.bitwise_and(p3, jnp.int32(-65536)),
                               jnp.float32))
            for l in range(16):
                es = ev[jnp.full((16,), l, jnp.int32)]
                i = g * 16 + l
                for j in range(D // 16):
                    rows[i, pl.ds(j * 16, 16)] = rows[i, pl.ds(j * 16, 16)] * es

        pltpu.sync_copy(rows, accum.at[mbuf.at[1]], add=True)

    def _iter(c, mbuf, rows, mbuf_n, rows_n):
        pltpu.make_async_copy(table_hbm.at[mbuf.at[0]], rows, semG).wait()

        @pl.when(c + 1 < CPT)
        def _():
            pltpu.make_async_copy(meta_hbm.at[mbase], mbuf_n, semM).wait()
            pltpu.async_copy(table_hbm.at[mbuf_n.at[0]], rows_n, semG)

        _do_chunk(mbuf, rows)

        @pl.when(c + 2 < CPT)
        def _():
            pltpu.async_copy(meta_hbm.at[mbase + c + 2], mbuf, semM)

    def _pair(cc, _):
        _iter(2 * cc, mbufA, rowsA, mbufB, rowsB)
        _iter(2 * cc + 1, mbufB, rowsB, mbufA, rowsA)
        return 0

    lax.fori_loop(0, CPT // 2, _pair, 0)
    plsc.subcore_barrier()

    pltpu.sync_copy(accum.at[pl.ds(sid * RPT, RPT)],
                    out_hbm.at[cid, pl.ds(sid * RPT, RPT)])


def _sc(table, meta, s13):
    mesh = plsc.VectorSubcoreMesh(core_axis_name="c", subcore_axis_name="s")
    f = functools.partial(
        pl.kernel,
        out_type=jax.ShapeDtypeStruct((NC, NA, D), jnp.float32),
        mesh=mesh,
        scratch_types=[
            pltpu.VMEM((3, CH), jnp.int32),
            pltpu.VMEM((3, CH), jnp.int32),
            pltpu.VMEM((N,), jnp.int32),
            pltpu.VMEM((CH, D), jnp.float32),
            pltpu.VMEM((CH, D), jnp.float32),
            pltpu.VMEM_SHARED((NA, D), jnp.float32),
            pltpu.SemaphoreType.DMA,
            pltpu.SemaphoreType.DMA,
        ],
        compiler_params=pltpu.CompilerParams(needs_layout_passes=False),
    )(_sc_body)
    return f(table, meta, s13)


def _t3_body(p0_ref, p1_ref, o_ref):
    o_ref[...] = p0_ref[...] + p1_ref[...]


def _t3(p0, p1):
    return pl.pallas_call(
        _t3_body,
        grid=(N // BN,),
        in_specs=[pl.BlockSpec((BN, D), lambda i: (i, 0)),
                  pl.BlockSpec((BN, D), lambda i: (i, 0))],
        out_specs=pl.BlockSpec((BN, D), lambda i: (i, 0)),
        out_shape=jax.ShapeDtypeStruct((N, D), jnp.float32),
    )(p0, p1)


def kernel(x, edge_index, edge_attr, rel_type, weight, shared_W, attn_W):
    src = edge_index[0]
    dst = edge_index[1]

    y, s = _t1(x, weight, shared_W, attn_W)
    table = y.reshape(N * R, D)
    s1u = lax.bitcast_convert_type(s[:, 0].astype(jnp.bfloat16),
                                   jnp.uint16).astype(jnp.uint32)
    s3u = lax.bitcast_convert_type(s[:, 2].astype(jnp.bfloat16),
                                   jnp.uint16).astype(jnp.uint32)
    s13 = lax.bitcast_convert_type(s1u | (s3u << 16), jnp.int32)

    t = _t2(edge_attr, shared_W, attn_W).reshape(EP)

    pad = EP - E
    gidx = src * R + rel_type
    gidx2 = jnp.concatenate([gidx, jnp.zeros((pad,), jnp.int32)]).reshape(
        NW * CPT, CH)
    dst2 = jnp.concatenate([dst, jnp.full((pad,), N, jnp.int32)]).reshape(
        NW * CPT, CH)
    tbits = lax.bitcast_convert_type(t, jnp.int32).reshape(NW * CPT, CH)
    meta = jnp.stack([gidx2, dst2, tbits], axis=1)

    partial = _sc(table, meta, s13)
    return _t3(partial[0], partial[1])


# speedup vs baseline: 1.2254x; 1.0429x over previous
import functools
import jax
import jax.numpy as jnp
from jax import lax
from jax.experimental import pallas as pl
from jax.experimental.pallas import tpu as pltpu
from jax.experimental.pallas import tpu_sc as plsc

N = 10000
E = 160000
D = 128
R = 8

NC = 2
NS = 16
NW = NC * NS
CH = 128
CPT = 40
EPT = CH * CPT
EP = EPT * NW
NA = 10240
RPT = NA // NS

BN = 1000


def _t1_body(x_ref, w_ref, sw_ref, aw_ref, y_ref, s_ref):
    xb = x_ref[...]
    a = aw_ref[...].reshape(3, D)
    v = jnp.dot(a, sw_ref[...], preferred_element_type=jnp.float32)
    vpad = jnp.concatenate([v, jnp.zeros((D - 3, D), jnp.float32)], axis=0)
    s_ref[...] = jnp.dot(xb, vpad.T, preferred_element_type=jnp.float32)
    for r in range(R):
        y_ref[:, r, :] = jnp.dot(xb, w_ref[r], preferred_element_type=jnp.float32)


def _t1(x, weight, shared_W, attn_W):
    return pl.pallas_call(
        _t1_body,
        grid=(N // BN,),
        in_specs=[
            pl.BlockSpec((BN, D), lambda i: (i, 0)),
            pl.BlockSpec((R, D, D), lambda i: (0, 0, 0)),
            pl.BlockSpec((D, D), lambda i: (0, 0)),
            pl.BlockSpec((1, 3 * D), lambda i: (0, 0)),
        ],
        out_specs=[
            pl.BlockSpec((BN, R, D), lambda i: (i, 0, 0)),
            pl.BlockSpec((BN, D), lambda i: (i, 0)),
        ],
        out_shape=[
            jax.ShapeDtypeStruct((N, R, D), jnp.float32),
            jax.ShapeDtypeStruct((N, D), jnp.float32),
        ],
    )(x, weight, shared_W, attn_W)


_T2_ROWS = 625
_T2_OUT_ROWS = EP // 256
_T2_B = 48


def _t2_body(ea_ref, sw_ref, aw_ref, t_ref):
    a = aw_ref[...].reshape(3, D)
    v = jnp.dot(a, sw_ref[...], preferred_element_type=jnp.float32)
    v2 = v[1]
    eb = ea_ref[...]
    t_ref[...] = jnp.sum(eb * v2[None, None, :], axis=-1)


def _t2(edge_attr, shared_W, attn_W):
    ea3 = edge_attr.reshape(_T2_ROWS, 256, D)
    return pl.pallas_call(
        _t2_body,
        grid=(pl.cdiv(_T2_ROWS, _T2_B),),
        in_specs=[
            pl.BlockSpec((_T2_B, 256, D), lambda i: (i, 0, 0)),
            pl.BlockSpec((D, D), lambda i: (0, 0)),
            pl.BlockSpec((1, 3 * D), lambda i: (0, 0)),
        ],
        out_specs=pl.BlockSpec((_T2_B, 256), lambda i: (i, 0)),
        out_shape=jax.ShapeDtypeStruct((_T2_OUT_ROWS, 256), jnp.float32),
    )(ea3, shared_W, attn_W)


def _sc_body(table_hbm, meta_hbm, s13_hbm, out_hbm,
             mbufA, mbufB, s13_v, rowsA, rowsB, accum, semG, semM):
    cid = lax.axis_index("c")
    sid = lax.axis_index("s")
    wid = sid * NC + cid
    mbase = wid * CPT

    pltpu.sync_copy(s13_hbm, s13_v)

    zero = jnp.zeros((16,), jnp.float32)

    def _zrow(i, _):
        for j in range(D // 16):
            rowsA[i, pl.ds(j * 16, 16)] = zero
        return 0

    lax.fori_loop(0, CH, _zrow, 0)
    for k in range(RPT // CH):
        pltpu.sync_copy(rowsA, accum.at[pl.ds(sid * RPT + k * CH, CH)])
    plsc.subcore_barrier()

    pltpu.sync_copy(meta_hbm.at[mbase], mbufA)
    pltpu.async_copy(table_hbm.at[mbufA.at[0]], rowsA, semG)
    pltpu.async_copy(meta_hbm.at[mbase + 1], mbufB, semM)

    def _do_chunk(mbuf, rows):
        @plsc.parallel_loop(0, CH // 16, unroll=2)
        def _grp(g):
            gv = mbuf[0, pl.ds(g * 16, 16)]
            dstv = mbuf[1, pl.ds(g * 16, 16)]
            tv = plsc.bitcast(mbuf[2, pl.ds(g * 16, 16)], jnp.float32)
            p1 = plsc.load_gather(s13_v, [lax.shift_right_logical(gv, 3)])
            p3 = plsc.load_gather(s13_v, [dstv])
            ev = (plsc.bitcast(lax.shift_left(p1, 16), jnp.float32) + tv +
                  plsc.bitcast(jnp.bitwise_and(p3, jnp.int32(-65536)),
                               jnp.float32))
            for l in range(16):
                es = ev[jnp.full((16,), l, jnp.int32)]
                i = g * 16 + l
                for j in range(D // 16):
                    rows[i, pl.ds(j * 16, 16)] = rows[i, pl.ds(j * 16, 16)] * es

        pltpu.sync_copy(rows, accum.at[mbuf.at[1]], add=True)

    def _iter(c, mbuf, rows, mbuf_n, rows_n):
        pltpu.make_async_copy(table_hbm.at[mbuf.at[0]], rows, semG).wait()

        @pl.when(c + 1 < CPT)
        def _():
            pltpu.make_async_copy(meta_hbm.at[mbase], mbuf_n, semM).wait()
            pltpu.async_copy(table_hbm.at[mbuf_n.at[0]], rows_n, semG)

        _do_chunk(mbuf, rows)

        @pl.when(c + 2 < CPT)
        def _():
            pltpu.async_copy(meta_hbm.at[mbase + c + 2], mbuf, semM)

    def _pair(cc, _):
        _iter(2 * cc, mbufA, rowsA, mbufB, rowsB)
        _iter(2 * cc + 1, mbufB, rowsB, mbufA, rowsA)
        return 0

    lax.fori_loop(0, CPT // 2, _pair, 0)
    plsc.subcore_barrier()

    pltpu.sync_copy(accum.at[pl.ds(sid * RPT, RPT)],
                    out_hbm.at[cid, pl.ds(sid * RPT, RPT)])


def _sc(table, meta, s13):
    mesh = plsc.VectorSubcoreMesh(core_axis_name="c", subcore_axis_name="s")
    f = functools.partial(
        pl.kernel,
        out_type=jax.ShapeDtypeStruct((NC, NA, D), jnp.float32),
        mesh=mesh,
        scratch_types=[
            pltpu.VMEM((3, CH), jnp.int32),
            pltpu.VMEM((3, CH), jnp.int32),
            pltpu.VMEM((N,), jnp.int32),
            pltpu.VMEM((CH, D), jnp.float32),
            pltpu.VMEM((CH, D), jnp.float32),
            pltpu.VMEM_SHARED((NA, D), jnp.float32),
            pltpu.SemaphoreType.DMA,
            pltpu.SemaphoreType.DMA,
        ],
        compiler_params=pltpu.CompilerParams(needs_layout_passes=False),
    )(_sc_body)
    return f(table, meta, s13)


def _t3_body(p0_ref, p1_ref, o_ref):
    o_ref[...] = p0_ref[...] + p1_ref[...]


def _t3(p0, p1):
    return pl.pallas_call(
        _t3_body,
        grid=(N // BN,),
        in_specs=[pl.BlockSpec((BN, D), lambda i: (i, 0)),
                  pl.BlockSpec((BN, D), lambda i: (i, 0))],
        out_specs=pl.BlockSpec((BN, D), lambda i: (i, 0)),
        out_shape=jax.ShapeDtypeStruct((N, D), jnp.float32),
    )(p0, p1)


def kernel(x, edge_index, edge_attr, rel_type, weight, shared_W, attn_W):
    src = edge_index[0]
    dst = edge_index[1]

    y, s = _t1(x, weight, shared_W, attn_W)
    table = y.reshape(N * R, D)
    s1u = lax.bitcast_convert_type(s[:, 0].astype(jnp.bfloat16),
                                   jnp.uint16).astype(jnp.uint32)
    s3u = lax.bitcast_convert_type(s[:, 2].astype(jnp.bfloat16),
                                   jnp.uint16).astype(jnp.uint32)
    s13 = lax.bitcast_convert_type(s1u | (s3u << 16), jnp.int32)

    t = _t2(edge_attr, shared_W, attn_W).reshape(EP)

    pad = EP - E
    gidx = src * R + rel_type
    gidx2 = jnp.concatenate([gidx, jnp.zeros((pad,), jnp.int32)]).reshape(
        NW * CPT, CH)
    dst2 = jnp.concatenate([dst, jnp.full((pad,), N, jnp.int32)]).reshape(
        NW * CPT, CH)
    tbits = lax.bitcast_convert_type(t, jnp.int32).reshape(NW * CPT, CH)
    meta = jnp.stack([gidx2, dst2, tbits], axis=1)

    partial = _sc(table, meta, s13)
    return _t3(partial[0], partial[1])


# speedup vs baseline: 1.2514x; 1.0212x over previous
import functools
import jax
import jax.numpy as jnp
from jax import lax
from jax.experimental import pallas as pl
from jax.experimental.pallas import tpu as pltpu
from jax.experimental.pallas import tpu_sc as plsc

N = 10000
E = 160000
D = 128
R = 8

NC = 2
NS = 16
NW = NC * NS
CH = 128
CPT = 40
EPT = CH * CPT
EP = EPT * NW
NA = 10240
RPT = NA // NS

BN = 2000


def _t1_body(x_ref, w_ref, sw_ref, aw_ref, y_ref, s_ref):
    xb = x_ref[...]
    a = aw_ref[...].reshape(3, D)
    v = jnp.dot(a, sw_ref[...], preferred_element_type=jnp.float32)
    vpad = jnp.concatenate([v, jnp.zeros((D - 3, D), jnp.float32)], axis=0)
    s_ref[...] = jnp.dot(xb, vpad.T, preferred_element_type=jnp.float32)
    for r in range(R):
        y_ref[:, r, :] = jnp.dot(xb, w_ref[r], preferred_element_type=jnp.float32)


def _t1(x, weight, shared_W, attn_W):
    return pl.pallas_call(
        _t1_body,
        grid=(N // BN,),
        in_specs=[
            pl.BlockSpec((BN, D), lambda i: (i, 0)),
            pl.BlockSpec((R, D, D), lambda i: (0, 0, 0)),
            pl.BlockSpec((D, D), lambda i: (0, 0)),
            pl.BlockSpec((1, 3 * D), lambda i: (0, 0)),
        ],
        out_specs=[
            pl.BlockSpec((BN, R, D), lambda i: (i, 0, 0)),
            pl.BlockSpec((BN, D), lambda i: (i, 0)),
        ],
        out_shape=[
            jax.ShapeDtypeStruct((N, R, D), jnp.float32),
            jax.ShapeDtypeStruct((N, D), jnp.float32),
        ],
    )(x, weight, shared_W, attn_W)


_T2_ROWS = 625
_T2_OUT_ROWS = EP // 256
_T2_B = 80


def _t2_body(ea_ref, sw_ref, aw_ref, t_ref):
    a = aw_ref[...].reshape(3, D)
    v = jnp.dot(a, sw_ref[...], preferred_element_type=jnp.float32)
    v2 = v[1]
    eb = ea_ref[...]
    t_ref[...] = jnp.sum(eb * v2[None, None, :], axis=-1)


def _t2(edge_attr, shared_W, attn_W):
    ea3 = edge_attr.reshape(_T2_ROWS, 256, D)
    return pl.pallas_call(
        _t2_body,
        grid=(pl.cdiv(_T2_ROWS, _T2_B),),
        in_specs=[
            pl.BlockSpec((_T2_B, 256, D), lambda i: (i, 0, 0)),
            pl.BlockSpec((D, D), lambda i: (0, 0)),
            pl.BlockSpec((1, 3 * D), lambda i: (0, 0)),
        ],
        out_specs=pl.BlockSpec((_T2_B, 256), lambda i: (i, 0)),
        out_shape=jax.ShapeDtypeStruct((_T2_OUT_ROWS, 256), jnp.float32),
    )(ea3, shared_W, attn_W)


def _sc_body(table_hbm, meta_hbm, s13_hbm, out_hbm,
             mbufA, mbufB, s13_v, rowsA, rowsB, accum, semG, semM):
    cid = lax.axis_index("c")
    sid = lax.axis_index("s")
    wid = sid * NC + cid
    mbase = wid * CPT

    pltpu.sync_copy(s13_hbm, s13_v)

    zero = jnp.zeros((16,), jnp.float32)

    def _zrow(i, _):
        for j in range(D // 16):
            rowsA[i, pl.ds(j * 16, 16)] = zero
        return 0

    lax.fori_loop(0, CH, _zrow, 0)
    for k in range(RPT // CH):
        pltpu.sync_copy(rowsA, accum.at[pl.ds(sid * RPT + k * CH, CH)])
    plsc.subcore_barrier()

    pltpu.sync_copy(meta_hbm.at[mbase], mbufA)
    pltpu.async_copy(table_hbm.at[mbufA.at[0]], rowsA, semG)
    pltpu.async_copy(meta_hbm.at[mbase + 1], mbufB, semM)

    def _do_chunk(mbuf, rows):
        @plsc.parallel_loop(0, CH // 16, unroll=2)
        def _grp(g):
            gv = mbuf[0, pl.ds(g * 16, 16)]
            dstv = mbuf[1, pl.ds(g * 16, 16)]
            tv = plsc.bitcast(mbuf[2, pl.ds(g * 16, 16)], jnp.float32)
            p1 = plsc.load_gather(s13_v, [lax.shift_right_logical(gv, 3)])
            p3 = plsc.load_gather(s13_v, [dstv])
            ev = (plsc.bitcast(lax.shift_left(p1, 16), jnp.float32) + tv +
                  plsc.bitcast(jnp.bitwise_and(p3, jnp.int32(-65536)),
                               jnp.float32))
            for l in range(16):
                es = ev[jnp.full((16,), l, jnp.int32)]
                i = g * 16 + l
                for j in range(D // 16):
                    rows[i, pl.ds(j * 16, 16)] = rows[i, pl.ds(j * 16, 16)] * es

        pltpu.sync_copy(rows, accum.at[mbuf.at[1]], add=True)

    def _iter(c, mbuf, rows, mbuf_n, rows_n):
        pltpu.make_async_copy(table_hbm.at[mbuf.at[0]], rows, semG).wait()

        @pl.when(c + 1 < CPT)
        def _():
            pltpu.make_async_copy(meta_hbm.at[mbase], mbuf_n, semM).wait()
            pltpu.async_copy(table_hbm.at[mbuf_n.at[0]], rows_n, semG)

        _do_chunk(mbuf, rows)

        @pl.when(c + 2 < CPT)
        def _():
            pltpu.async_copy(meta_hbm.at[mbase + c + 2], mbuf, semM)

    def _pair(cc, _):
        _iter(2 * cc, mbufA, rowsA, mbufB, rowsB)
        _iter(2 * cc + 1, mbufB, rowsB, mbufA, rowsA)
        return 0

    lax.fori_loop(0, CPT // 2, _pair, 0)
    plsc.subcore_barrier()

    pltpu.sync_copy(accum.at[pl.ds(sid * RPT, RPT)],
                    out_hbm.at[cid, pl.ds(sid * RPT, RPT)])


def _sc(table, meta, s13):
    mesh = plsc.VectorSubcoreMesh(core_axis_name="c", subcore_axis_name="s")
    f = functools.partial(
        pl.kernel,
        out_type=jax.ShapeDtypeStruct((NC, NA, D), jnp.float32),
        mesh=mesh,
        scratch_types=[
            pltpu.VMEM((3, CH), jnp.int32),
            pltpu.VMEM((3, CH), jnp.int32),
            pltpu.VMEM((N,), jnp.int32),
            pltpu.VMEM((CH, D), jnp.float32),
            pltpu.VMEM((CH, D), jnp.float32),
            pltpu.VMEM_SHARED((NA, D), jnp.float32),
            pltpu.SemaphoreType.DMA,
            pltpu.SemaphoreType.DMA,
        ],
        compiler_params=pltpu.CompilerParams(needs_layout_passes=False),
    )(_sc_body)
    return f(table, meta, s13)


def _t3_body(p0_ref, p1_ref, o_ref):
    o_ref[...] = p0_ref[...] + p1_ref[...]


def _t3(p0, p1):
    return pl.pallas_call(
        _t3_body,
        grid=(N // BN,),
        in_specs=[pl.BlockSpec((BN, D), lambda i: (i, 0)),
                  pl.BlockSpec((BN, D), lambda i: (i, 0))],
        out_specs=pl.BlockSpec((BN, D), lambda i: (i, 0)),
        out_shape=jax.ShapeDtypeStruct((N, D), jnp.float32),
    )(p0, p1)


def kernel(x, edge_index, edge_attr, rel_type, weight, shared_W, attn_W):
    src = edge_index[0]
    dst = edge_index[1]

    y, s = _t1(x, weight, shared_W, attn_W)
    table = y.reshape(N * R, D)
    s1u = lax.bitcast_convert_type(s[:, 0].astype(jnp.bfloat16),
                                   jnp.uint16).astype(jnp.uint32)
    s3u = lax.bitcast_convert_type(s[:, 2].astype(jnp.bfloat16),
                                   jnp.uint16).astype(jnp.uint32)
    s13 = lax.bitcast_convert_type(s1u | (s3u << 16), jnp.int32)

    t = _t2(edge_attr, shared_W, attn_W).reshape(EP)

    pad = EP - E
    gidx = src * R + rel_type
    gidx2 = jnp.concatenate([gidx, jnp.zeros((pad,), jnp.int32)]).reshape(
        NW * CPT, CH)
    dst2 = jnp.concatenate([dst, jnp.full((pad,), N, jnp.int32)]).reshape(
        NW * CPT, CH)
    tbits = lax.bitcast_convert_type(t, jnp.int32).reshape(NW * CPT, CH)
    meta = jnp.stack([gidx2, dst2, tbits], axis=1)

    partial = _sc(table, meta, s13)
    return _t3(partial[0], partial[1])
